# baseline (speedup 1.0000x reference)
import functools

import jax
import jax.numpy as jnp
from jax import lax
from jax.experimental import pallas as pl
from jax.experimental.pallas import tpu as pltpu
from jax.experimental.pallas import tpu_sc as plsc

N_NODES = 10000
N_EDGES = 320000
D_FEAT = 128
LATENT = 32

NC = 2
NS = 16
NW = NC * NS

CHUNK = 128
EROWS = N_EDGES // CHUNK
NCHUNK = EROWS // NW
NEXTRA = EROWS - NCHUNK * NW
NBUF = 8
ROWS_PER_SUB = N_NODES // NS


def _sc_body(y_hbm, ei_hbm, zeros_hbm, out_hbm,
             src_v, dst_v, rows_v, zacc, y_sp,
             gsem0, gsem1, gsem2, gsem3, gsem4, gsem5, gsem6, gsem7,
             ssem0, ssem1, ssem2, ssem3, ssem4, ssem5, ssem6, ssem7):
    core = lax.axis_index("c")
    sid = lax.axis_index("s")
    w = core * NS + sid
    gsems = (gsem0, gsem1, gsem2, gsem3, gsem4, gsem5, gsem6, gsem7)
    ssems = (ssem0, ssem1, ssem2, ssem3, ssem4, ssem5, ssem6, ssem7)

    sl = pl.ds(sid * ROWS_PER_SUB, ROWS_PER_SUB)
    pltpu.sync_copy(zeros_hbm.at[sl], zacc.at[sl])
    pltpu.sync_copy(y_hbm.at[sl], y_sp.at[sl])

    pltpu.sync_copy(ei_hbm.at[0, pl.ds(w * NCHUNK, NCHUNK)],
                    src_v.at[pl.ds(0, NCHUNK)])
    pltpu.sync_copy(ei_hbm.at[1, pl.ds(w * NCHUNK, NCHUNK)],
                    dst_v.at[pl.ds(0, NCHUNK)])

    @pl.when(w < NEXTRA)
    def _():
        pltpu.sync_copy(ei_hbm.at[0, pl.ds(NCHUNK * NW + w, 1)],
                        src_v.at[pl.ds(NCHUNK, 1)])
        pltpu.sync_copy(ei_hbm.at[1, pl.ds(NCHUNK * NW + w, 1)],
                        dst_v.at[pl.ds(NCHUNK, 1)])

    plsc.subcore_barrier()

    def gather_start(j, b):
        pltpu.async_copy(y_sp.at[src_v.at[j]], rows_v.at[b], gsems[b])

    def gather_wait(j, b):
        pltpu.make_async_copy(
            y_sp.at[src_v.at[j]], rows_v.at[b], gsems[b]).wait()

    def scat_start(j, b):
        pltpu.async_copy(rows_v.at[b], zacc.at[dst_v.at[j]], ssems[b],
                         add=True)

    def scat_wait(j, b):
        pltpu.make_async_copy(rows_v.at[b], zacc.at[dst_v.at[j]],
                              ssems[b]).wait()

    for b in range(NBUF - 1):
        gather_start(b, b)

    def static_step(j):
        b = j % NBUF
        gather_wait(j, b)
        scat_start(j, b)
        k = j + NBUF - 1
        if k < NCHUNK:
            bk = (b - 1) % NBUF
            if j >= 1:
                scat_wait(j - 1, bk)
            gather_start(k, bk)

    for j in range(NBUF):
        static_step(j)

    def body(g, carry):
        for b in range(NBUF):
            j = g * NBUF + b
            gather_wait(j, b)
            scat_start(j, b)
            bk = (b - 1) % NBUF
            scat_wait(j - 1, bk)
            gather_start(j + NBUF - 1, bk)
        return carry

    _G_END = (NCHUNK - 2 * NBUF) // NBUF + 1
    lax.fori_loop(1, _G_END, body, 0)

    for j in range(_G_END * NBUF, NCHUNK):
        static_step(j)
    for j in range(NCHUNK - NBUF, NCHUNK):
        scat_wait(j, j % NBUF)

    @pl.when(w < NEXTRA)
    def _():
        pltpu.async_copy(
            y_sp.at[src_v.at[NCHUNK]], rows_v.at[0], gsems[0]).wait()
        pltpu.sync_copy(rows_v.at[0], zacc.at[dst_v.at[NCHUNK]], add=True)

    plsc.subcore_barrier()
    pltpu.sync_copy(zacc.at[sl], out_hbm.at[core, sl])


_sc_scatter = functools.partial(
    pl.kernel,
    out_type=jax.ShapeDtypeStruct((NC, N_NODES, LATENT), jnp.float32),
    mesh=plsc.VectorSubcoreMesh(core_axis_name="c", subcore_axis_name="s"),
    scratch_types=[
        pltpu.VMEM((NCHUNK + 1, CHUNK), jnp.int32),
        pltpu.VMEM((NCHUNK + 1, CHUNK), jnp.int32),
        pltpu.VMEM((NBUF, CHUNK, LATENT), jnp.float32),
        pltpu.VMEM_SHARED((N_NODES, LATENT), jnp.float32),
        pltpu.VMEM_SHARED((N_NODES, LATENT), jnp.float32),
    ] + [pltpu.SemaphoreType.DMA] * 16,
    compiler_params=pltpu.CompilerParams(use_tc_tiling_on_sc=False),
)(_sc_body)


def _enc_body(x_ref, w_ref, y_ref):
    y_ref[...] = jnp.dot(x_ref[...], w_ref[...],
                         preferred_element_type=jnp.float32)


def _tail_body(zp_ref, benc_ref, wdec_ref, bdec_ref, uw_ref,
               lat_ref, rec_ref, util_ref, prob_ref):
    z = zp_ref[0] + zp_ref[1] + benc_ref[...]
    lat = jnp.maximum(z, 0.0)
    lat_ref[...] = lat
    rec = jnp.dot(lat, wdec_ref[...],
                  preferred_element_type=jnp.float32) + bdec_ref[...]
    rec_ref[...] = rec
    util_ref[...] = jnp.dot(rec, uw_ref[...],
                            preferred_element_type=jnp.float32)
    e = jnp.exp(rec)
    prob_ref[...] = e / jnp.sum(e, axis=1, keepdims=True)


_RB = 1000


def kernel(x, edge_index, W_enc, b_enc, W_dec, b_dec, utility_w):
    f32 = jnp.float32

    y = pl.pallas_call(
        _enc_body,
        grid=(N_NODES // _RB,),
        in_specs=[
            pl.BlockSpec((_RB, D_FEAT), lambda i: (i, 0)),
            pl.BlockSpec((D_FEAT, LATENT), lambda i: (0, 0)),
        ],
        out_specs=pl.BlockSpec((_RB, LATENT), lambda i: (i, 0)),
        out_shape=jax.ShapeDtypeStruct((N_NODES, LATENT), f32),
    )(x, W_enc)

    ei = edge_index.reshape(2, EROWS, CHUNK)
    zeros = jnp.zeros((N_NODES, LATENT), f32)
    partials = _sc_scatter(y, ei, zeros)

    uw_pad = jnp.concatenate(
        [utility_w, jnp.zeros((D_FEAT - 2,), f32)]).reshape(D_FEAT, 1)
    latent, recon, util2d, prob = pl.pallas_call(
        _tail_body,
        grid=(N_NODES // _RB,),
        in_specs=[
            pl.BlockSpec((NC, _RB, LATENT), lambda i: (0, i, 0)),
            pl.BlockSpec((1, LATENT), lambda i: (0, 0)),
            pl.BlockSpec((LATENT, D_FEAT), lambda i: (0, 0)),
            pl.BlockSpec((1, D_FEAT), lambda i: (0, 0)),
            pl.BlockSpec((D_FEAT, 1), lambda i: (0, 0)),
        ],
        out_specs=[
            pl.BlockSpec((_RB, LATENT), lambda i: (i, 0)),
            pl.BlockSpec((_RB, D_FEAT), lambda i: (i, 0)),
            pl.BlockSpec((_RB, 1), lambda i: (i, 0)),
            pl.BlockSpec((_RB, D_FEAT), lambda i: (i, 0)),
        ],
        out_shape=[
            jax.ShapeDtypeStruct((N_NODES, LATENT), f32),
            jax.ShapeDtypeStruct((N_NODES, D_FEAT), f32),
            jax.ShapeDtypeStruct((N_NODES, 1), f32),
            jax.ShapeDtypeStruct((N_NODES, D_FEAT), f32),
        ],
    )(partials, b_enc.reshape(1, LATENT), W_dec, b_dec.reshape(1, D_FEAT),
      uw_pad)

    return latent, recon, util2d.reshape(N_NODES), prob


# speedup vs baseline: 1.0507x; 1.0507x over previous
import functools

import jax
import jax.numpy as jnp
from jax import lax
from jax.experimental import pallas as pl
from jax.experimental.pallas import tpu as pltpu
from jax.experimental.pallas import tpu_sc as plsc

N_NODES = 10000
N_EDGES = 320000
D_FEAT = 128
LATENT = 32

NC = 2
NS = 16
NW = NC * NS

CHUNK = 128
EROWS = N_EDGES // CHUNK
NCHUNK = EROWS // NW
NEXTRA = EROWS - NCHUNK * NW
NBUF = 8
ROWS_PER_SUB = N_NODES // NS


def _sc_body(y_hbm, ei_hbm, zeros_hbm, out_hbm,
             src_v, dst_v, rows_v, zacc, y_sp,
             gsem0, gsem1, gsem2, gsem3, gsem4, gsem5, gsem6, gsem7,
             ssem0, ssem1, ssem2, ssem3, ssem4, ssem5, ssem6, ssem7):
    core = lax.axis_index("c")
    sid = lax.axis_index("s")
    w = core * NS + sid
    gsems = (gsem0, gsem1, gsem2, gsem3, gsem4, gsem5, gsem6, gsem7)
    ssems = (ssem0, ssem1, ssem2, ssem3, ssem4, ssem5, ssem6, ssem7)

    sl = pl.ds(sid * ROWS_PER_SUB, ROWS_PER_SUB)
    pltpu.sync_copy(zeros_hbm.at[sl], zacc.at[sl])
    pltpu.sync_copy(y_hbm.at[sl], y_sp.at[sl])

    pltpu.sync_copy(ei_hbm.at[0, pl.ds(w * NCHUNK, NCHUNK)],
                    src_v.at[pl.ds(0, NCHUNK)])
    pltpu.sync_copy(ei_hbm.at[1, pl.ds(w * NCHUNK, NCHUNK)],
                    dst_v.at[pl.ds(0, NCHUNK)])

    @pl.when(w < NEXTRA)
    def _():
        pltpu.sync_copy(ei_hbm.at[0, pl.ds(NCHUNK * NW + w, 1)],
                        src_v.at[pl.ds(NCHUNK, 1)])
        pltpu.sync_copy(ei_hbm.at[1, pl.ds(NCHUNK * NW + w, 1)],
                        dst_v.at[pl.ds(NCHUNK, 1)])

    plsc.subcore_barrier()

    def gather_start(j, b):
        pltpu.async_copy(y_sp.at[src_v.at[j]], rows_v.at[b], gsems[b])

    def gather_wait(j, b):
        pltpu.make_async_copy(
            y_sp.at[src_v.at[j]], rows_v.at[b], gsems[b]).wait()

    def scat_start(j, b):
        pltpu.async_copy(rows_v.at[b], zacc.at[dst_v.at[j]], ssems[b],
                         add=True)

    def scat_wait(j, b):
        pltpu.make_async_copy(rows_v.at[b], zacc.at[dst_v.at[j]],
                              ssems[b]).wait()

    for b in range(NBUF - 1):
        gather_start(b, b)

    def static_step(j):
        b = j % NBUF
        gather_wait(j, b)
        scat_start(j, b)
        k = j + NBUF - 1
        if k < NCHUNK:
            bk = (b - 1) % NBUF
            if j >= 1:
                scat_wait(j - 1, bk)
            gather_start(k, bk)

    for j in range(NBUF):
        static_step(j)

    def body(g, carry):
        for b in range(NBUF):
            j = g * NBUF + b
            gather_wait(j, b)
            scat_start(j, b)
            bk = (b - 1) % NBUF
            scat_wait(j - 1, bk)
            gather_start(j + NBUF - 1, bk)
        return carry

    _G_END = (NCHUNK - 2 * NBUF) // NBUF + 1
    lax.fori_loop(1, _G_END, body, 0)

    for j in range(_G_END * NBUF, NCHUNK):
        static_step(j)
    for j in range(NCHUNK - NBUF, NCHUNK):
        scat_wait(j, j % NBUF)

    @pl.when(w < NEXTRA)
    def _():
        pltpu.async_copy(
            y_sp.at[src_v.at[NCHUNK]], rows_v.at[0], gsems[0]).wait()
        pltpu.sync_copy(rows_v.at[0], zacc.at[dst_v.at[NCHUNK]], add=True)

    plsc.subcore_barrier()
    pltpu.sync_copy(zacc.at[sl], out_hbm.at[core, sl])


_sc_scatter = functools.partial(
    pl.kernel,
    out_type=jax.ShapeDtypeStruct((NC, N_NODES, LATENT), jnp.float32),
    mesh=plsc.VectorSubcoreMesh(core_axis_name="c", subcore_axis_name="s"),
    scratch_types=[
        pltpu.VMEM((NCHUNK + 1, CHUNK), jnp.int32),
        pltpu.VMEM((NCHUNK + 1, CHUNK), jnp.int32),
        pltpu.VMEM((NBUF, CHUNK, LATENT), jnp.float32),
        pltpu.VMEM_SHARED((N_NODES, LATENT), jnp.float32),
        pltpu.VMEM_SHARED((N_NODES, LATENT), jnp.float32),
    ] + [pltpu.SemaphoreType.DMA] * 16,
    compiler_params=pltpu.CompilerParams(use_tc_tiling_on_sc=False),
)(_sc_body)


def _enc_body(x_ref, w_ref, y_ref):
    y_ref[...] = jnp.dot(x_ref[...], w_ref[...],
                         preferred_element_type=jnp.float32)


def _tail_body(zp_ref, benc_ref, wdec_ref, bdec_ref, uw_ref,
               lat_ref, rec_ref, util_ref, prob_ref):
    z = zp_ref[0] + zp_ref[1] + benc_ref[...]
    lat = jnp.maximum(z, 0.0)
    lat_ref[...] = lat
    rec = jnp.dot(lat, wdec_ref[...],
                  preferred_element_type=jnp.float32) + bdec_ref[...]
    rec_ref[...] = rec
    util_ref[...] = jnp.dot(rec, uw_ref[...],
                            preferred_element_type=jnp.float32)
    e = jnp.exp(rec)
    prob_ref[...] = e / jnp.sum(e, axis=1, keepdims=True)


_RB = 2000


def kernel(x, edge_index, W_enc, b_enc, W_dec, b_dec, utility_w):
    f32 = jnp.float32

    y = pl.pallas_call(
        _enc_body,
        grid=(N_NODES // _RB,),
        in_specs=[
            pl.BlockSpec((_RB, D_FEAT), lambda i: (i, 0)),
            pl.BlockSpec((D_FEAT, LATENT), lambda i: (0, 0)),
        ],
        out_specs=pl.BlockSpec((_RB, LATENT), lambda i: (i, 0)),
        out_shape=jax.ShapeDtypeStruct((N_NODES, LATENT), f32),
    )(x, W_enc)

    ei = edge_index.reshape(2, EROWS, CHUNK)
    zeros = jnp.zeros((N_NODES, LATENT), f32)
    partials = _sc_scatter(y, ei, zeros)

    uw_pad = jnp.concatenate(
        [utility_w, jnp.zeros((D_FEAT - 2,), f32)]).reshape(D_FEAT, 1)
    latent, recon, util2d, prob = pl.pallas_call(
        _tail_body,
        grid=(N_NODES // _RB,),
        in_specs=[
            pl.BlockSpec((NC, _RB, LATENT), lambda i: (0, i, 0)),
            pl.BlockSpec((1, LATENT), lambda i: (0, 0)),
            pl.BlockSpec((LATENT, D_FEAT), lambda i: (0, 0)),
            pl.BlockSpec((1, D_FEAT), lambda i: (0, 0)),
            pl.BlockSpec((D_FEAT, 1), lambda i: (0, 0)),
        ],
        out_specs=[
            pl.BlockSpec((_RB, LATENT), lambda i: (i, 0)),
            pl.BlockSpec((_RB, D_FEAT), lambda i: (i, 0)),
            pl.BlockSpec((_RB, 1), lambda i: (i, 0)),
            pl.BlockSpec((_RB, D_FEAT), lambda i: (i, 0)),
        ],
        out_shape=[
            jax.ShapeDtypeStruct((N_NODES, LATENT), f32),
            jax.ShapeDtypeStruct((N_NODES, D_FEAT), f32),
            jax.ShapeDtypeStruct((N_NODES, 1), f32),
            jax.ShapeDtypeStruct((N_NODES, D_FEAT), f32),
        ],
    )(partials, b_enc.reshape(1, LATENT), W_dec, b_dec.reshape(1, D_FEAT),
      uw_pad)

    return latent, recon, util2d.reshape(N_NODES), prob
